# baseline (speedup 1.0000x reference)
import functools

import jax
import jax.numpy as jnp
from jax import lax
from jax.experimental import pallas as pl
from jax.experimental.pallas import tpu as pltpu
from jax.experimental.pallas import tpu_sc as plsc

EMBED_DIM = 32
NUM_CORES = 2
NUM_SUBCORES = 16
NUM_WORKERS = NUM_CORES * NUM_SUBCORES
BLOCK = 128
NBUF = 8
DEPTH = 4
LANES = 16
TPAD = BLOCK + 1

_mesh = plsc.VectorSubcoreMesh(core_axis_name="c", subcore_axis_name="s")


@functools.partial(jax.jit, static_argnames=("hist",))
def _embed_lookup(xt3, table, hist):
    @functools.partial(
        pl.kernel,
        mesh=_mesh,
        out_type=jax.ShapeDtypeStruct(
            (hist, EMBED_DIM // 8, NUM_WORKERS, 8, BLOCK), jnp.float32),
        scratch_types=[
            pltpu.VMEM((hist, BLOCK), jnp.int32),
            pltpu.VMEM((NBUF, BLOCK, EMBED_DIM), jnp.bfloat16),
            pltpu.VMEM((NBUF, EMBED_DIM // 8, 8, TPAD), jnp.float32),
            pltpu.SemaphoreType.DMA((NBUF,)),
            pltpu.SemaphoreType.DMA((NBUF,)),
        ],
        compiler_params=pltpu.CompilerParams(
            use_tc_tiling_on_sc=False, needs_layout_passes=False),
    )
    def body(x_hbm, table_hbm, out_hbm, idx_v, rows_v, trows_v, gsem, wsem):
        wid = lax.axis_index("s") * NUM_CORES + lax.axis_index("c")
        pltpu.sync_copy(x_hbm.at[:, wid], idx_v)
        c_iota = lax.iota(jnp.int32, LANES)
        dd_vecs = []
        s_vecs = []
        for par in range(2):
            d_vec = 2 * c_iota + par
            dd_vecs.append(lax.shift_right_logical(d_vec, 3))
            s_vecs.append(lax.bitwise_and(d_vec, jnp.int32(7)))

        def gather(j, slot):
            return pltpu.make_async_copy(
                table_hbm.at[idx_v.at[j]], rows_v.at[slot], gsem.at[slot])

        def write(j, slot):
            return pltpu.make_async_copy(
                trows_v.at[slot, :, :, pl.ds(0, BLOCK)],
                out_hbm.at[j, :, wid], wsem.at[slot])

        def transpose(slot):
            trows = trows_v.at[slot]

            @plsc.parallel_loop(0, BLOCK, unroll=8)
            def _(c):
                csplat = lax.broadcast(c, (LANES,))
                row = rows_v[slot, c, :]
                evens_odds = plsc.unpack(
                    row, format=plsc.PackFormat.INTERLEAVED)
                for par in range(2):
                    plsc.store_scatter(
                        trows,
                        [dd_vecs[par], s_vecs[par], csplat],
                        evens_odds[par])

        for b in range(DEPTH):
            gather(b, b).start()

        def outer(g, carry):
            for b in range(NBUF):
                j = g * NBUF + b
                gather(j, b).wait()

                @pl.when(j >= NBUF)
                def _():
                    write(j - NBUF, b).wait()

                transpose(b)
                write(j, b).start()

                jn = j + DEPTH
                bn = (b + DEPTH) % NBUF

                @pl.when(jn < hist)
                def _():
                    gather(jn, bn).start()

            return carry

        lax.fori_loop(0, hist // NBUF, outer, 0)

        for b in range(NBUF):
            write(hist - NBUF + b, b).wait()

    return body(xt3, table)


def kernel(x, table):
    batch, hist = x.shape
    xt3 = jnp.transpose(x).astype(jnp.int32).reshape(
        hist, NUM_WORKERS, batch // NUM_WORKERS)
    out5 = _embed_lookup(xt3, table.astype(jnp.bfloat16), hist)
    return out5.transpose(2, 4, 0, 1, 3).reshape(batch, hist, EMBED_DIM)


# speedup vs baseline: 1.1737x; 1.1737x over previous
import functools

import jax
import jax.numpy as jnp
from jax import lax
from jax.experimental import pallas as pl
from jax.experimental.pallas import tpu as pltpu
from jax.experimental.pallas import tpu_sc as plsc

EMBED_DIM = 32
NUM_CORES = 2
NUM_SUBCORES = 16
NUM_WORKERS = NUM_CORES * NUM_SUBCORES
BLOCK = 128
NBUF = 8
DEPTH = 4
LANES = 16
TPAD = BLOCK + 1

_mesh = plsc.VectorSubcoreMesh(core_axis_name="c", subcore_axis_name="s")


@functools.partial(jax.jit, static_argnames=("hist",))
def _embed_lookup(xt3, table, hist):
    @functools.partial(
        pl.kernel,
        mesh=_mesh,
        out_type=jax.ShapeDtypeStruct(
            (hist, EMBED_DIM // 8, NUM_WORKERS, 8, BLOCK), jnp.float32),
        scratch_types=[
            pltpu.VMEM((hist, BLOCK), jnp.int32),
            pltpu.VMEM((NBUF, BLOCK, EMBED_DIM), jnp.float32),
            pltpu.VMEM((NBUF, EMBED_DIM // 8, 8, TPAD), jnp.float32),
            pltpu.SemaphoreType.DMA((NBUF,)),
            pltpu.SemaphoreType.DMA((NBUF,)),
        ],
        compiler_params=pltpu.CompilerParams(
            use_tc_tiling_on_sc=False, needs_layout_passes=False),
    )
    def body(x_hbm, table_hbm, out_hbm, idx_v, rows_v, trows_v, gsem, wsem):
        wid = lax.axis_index("s") * NUM_CORES + lax.axis_index("c")
        pltpu.sync_copy(x_hbm.at[:, wid], idx_v)
        c_iota = lax.iota(jnp.int32, LANES)
        dd_vecs = []
        s_vecs = []
        for half in range(EMBED_DIM // LANES):
            d_vec = c_iota + (half * LANES)
            dd_vecs.append(lax.shift_right_logical(d_vec, 3))
            s_vecs.append(lax.bitwise_and(d_vec, jnp.int32(7)))

        def gather(j, slot):
            return pltpu.make_async_copy(
                table_hbm.at[idx_v.at[j]], rows_v.at[slot], gsem.at[slot])

        def write(j, slot):
            return pltpu.make_async_copy(
                trows_v.at[slot, :, :, pl.ds(0, BLOCK)],
                out_hbm.at[j, :, wid], wsem.at[slot])

        def transpose(slot):
            trows = trows_v.at[slot]

            @plsc.parallel_loop(0, BLOCK, unroll=8)
            def _(c):
                csplat = lax.broadcast(c, (LANES,))
                for half in range(EMBED_DIM // LANES):
                    vals = rows_v[slot, c, pl.ds(half * LANES, LANES)]
                    plsc.store_scatter(
                        trows,
                        [dd_vecs[half], s_vecs[half], csplat],
                        vals)

        for b in range(DEPTH):
            gather(b, b).start()

        def outer(g, carry):
            for b in range(NBUF):
                j = g * NBUF + b
                gather(j, b).wait()

                @pl.when(j >= NBUF)
                def _():
                    write(j - NBUF, b).wait()

                transpose(b)
                write(j, b).start()

                jn = j + DEPTH
                bn = (b + DEPTH) % NBUF

                @pl.when(jn < hist)
                def _():
                    gather(jn, bn).start()

            return carry

        lax.fori_loop(0, hist // NBUF, outer, 0)

        for b in range(NBUF):
            write(hist - NBUF + b, b).wait()

    return body(xt3, table)


def kernel(x, table):
    batch, hist = x.shape
    xt3 = jnp.transpose(x).astype(jnp.int32).reshape(
        hist, NUM_WORKERS, batch // NUM_WORKERS)
    out5 = _embed_lookup(xt3, table, hist)
    return out5.transpose(2, 4, 0, 1, 3).reshape(batch, hist, EMBED_DIM)
